# baseline (speedup 1.0000x reference)
import jax
import jax.numpy as jnp
from jax.experimental import pallas as pl
from jax.experimental.pallas import tpu as pltpu

B = 16384
F_IN = 768
H = 1024
BM = 1024

_DN = (((1,), (1,)), ((), ()))


def _fused_kernel(xs_ref, xn_ref, w_ref, b_ref, ow_ref, ob_ref, o_ref):
    w8 = w_ref[...]
    b = b_ref[...]
    acc_s = jax.lax.dot_general(xs_ref[...].astype(jnp.float8_e4m3fn), w8, _DN,
                                preferred_element_type=jnp.float32) + b
    acc_n = jax.lax.dot_general(xn_ref[...].astype(jnp.float8_e4m3fn), w8, _DN,
                                preferred_element_type=jnp.float32) + b
    hs = jnp.square(jnp.clip(acc_s, 0.0, 1.0))
    hn = jnp.square(jnp.clip(acc_n, 0.0, 1.0))
    contrib = hs * ow_ref[0:1, :H] + hn * ow_ref[0:1, H:]
    logit = jnp.sum(contrib, axis=1, keepdims=True) + ob_ref[0]
    o_ref[...] = jax.nn.sigmoid(logit)


def kernel(stm_dense, nstm_dense, ft_w, ft_b, out_w, out_b):
    grid = (B // BM,)
    return pl.pallas_call(
        _fused_kernel,
        grid=grid,
        in_specs=[
            pl.BlockSpec((BM, F_IN), lambda i: (i, 0)),
            pl.BlockSpec((BM, F_IN), lambda i: (i, 0)),
            pl.BlockSpec((H, F_IN), lambda i: (0, 0)),
            pl.BlockSpec((H,), lambda i: (0,)),
            pl.BlockSpec((1, 2 * H), lambda i: (0, 0)),
            pl.BlockSpec((1,), lambda i: (0,)),
        ],
        out_specs=pl.BlockSpec((BM, 1), lambda i: (i, 0)),
        out_shape=jax.ShapeDtypeStruct((B, 1), jnp.float32),
        compiler_params=pltpu.CompilerParams(
            dimension_semantics=("parallel",),
        ),
    )(stm_dense, nstm_dense, ft_w.astype(jnp.float8_e4m3fn), ft_b, out_w, out_b)


# speedup vs baseline: 1.0403x; 1.0403x over previous
import jax
import jax.numpy as jnp
from jax.experimental import pallas as pl
from jax.experimental.pallas import tpu as pltpu

B = 16384
F_IN = 768
H = 1024
BM = 1024

_DN = (((1,), (1,)), ((), ()))


def _fused_kernel(xs_ref, xn_ref, w_ref, b_ref, ow_ref, ob_ref, o_ref):
    w8 = w_ref[...].astype(jnp.float8_e4m3fn)
    b = b_ref[...]
    acc_s = jax.lax.dot_general(xs_ref[...].astype(jnp.float8_e4m3fn), w8, _DN,
                                preferred_element_type=jnp.float32) + b
    acc_n = jax.lax.dot_general(xn_ref[...].astype(jnp.float8_e4m3fn), w8, _DN,
                                preferred_element_type=jnp.float32) + b
    hs = jnp.square(jnp.clip(acc_s, 0.0, 1.0))
    hn = jnp.square(jnp.clip(acc_n, 0.0, 1.0))
    contrib = hs * ow_ref[0:1, :H] + hn * ow_ref[0:1, H:]
    logit = jnp.sum(contrib, axis=1, keepdims=True) + ob_ref[0]
    o_ref[...] = jax.nn.sigmoid(logit)


def kernel(stm_dense, nstm_dense, ft_w, ft_b, out_w, out_b):
    grid = (B // BM,)
    return pl.pallas_call(
        _fused_kernel,
        grid=grid,
        in_specs=[
            pl.BlockSpec((BM, F_IN), lambda i: (i, 0)),
            pl.BlockSpec((BM, F_IN), lambda i: (i, 0)),
            pl.BlockSpec((H, F_IN), lambda i: (0, 0)),
            pl.BlockSpec((H,), lambda i: (0,)),
            pl.BlockSpec((1, 2 * H), lambda i: (0, 0)),
            pl.BlockSpec((1,), lambda i: (0,)),
        ],
        out_specs=pl.BlockSpec((BM, 1), lambda i: (i, 0)),
        out_shape=jax.ShapeDtypeStruct((B, 1), jnp.float32),
        compiler_params=pltpu.CompilerParams(
            dimension_semantics=("parallel",),
        ),
    )(stm_dense, nstm_dense, ft_w, ft_b, out_w, out_b)
